# baseline (speedup 1.0000x reference)
import functools

import jax
import jax.numpy as jnp
from jax import lax
from jax.experimental import pallas as pl
from jax.experimental.pallas import tpu as pltpu
from jax.experimental.pallas import tpu_sc as plsc

T = 10
NPF = 512
EPF = 4096
NIN = 256
EIN = 64
H = 256
NC = 8
HP = H // 2

NG = 5
TG = T // NG
EC = 2
ECHUNK = EPF // EC

NWORK = 32
GROWS = TG * EPF
RPW = GROWS // NWORK
CH = 128
NCHUNK = RPW // CH
NBUF = 3


def _ln(x, g, b):
    m = jnp.mean(x, axis=-1, keepdims=True)
    v = jnp.mean((x - m) ** 2, axis=-1, keepdims=True)
    return (x - m) * lax.rsqrt(v + 1e-5) * g + b


def _node_body(nf_ref, tpe_ref, Wn_ref, bn_ref, gn_ref, bn2_ref, o_ref):
    nf = nf_ref[...].reshape(T * NPF, NIN)
    node_h = _ln(jnp.dot(nf, Wn_ref[...], preferred_element_type=jnp.float32)
                 + bn_ref[...], gn_ref[...], bn2_ref[...])
    out = node_h.reshape(T, NPF, H) + tpe_ref[...]
    lo = out[:, :, :HP].astype(jnp.bfloat16).astype(jnp.float32)
    hi = out[:, :, HP:].astype(jnp.bfloat16).astype(jnp.float32)
    lo_u = lax.shift_right_logical(lax.bitcast_convert_type(lo, jnp.uint32),
                                   jnp.uint32(16))
    hi_u = lax.bitwise_and(lax.bitcast_convert_type(hi, jnp.uint32),
                           jnp.uint32(0xFFFF0000))
    o_ref[...] = lax.bitcast_convert_type(lax.bitwise_or(lo_u, hi_u), jnp.int32)


def _sc_gather(table_hbm, isrc_hbm, idst_hbm, gs_hbm, gd_hbm,
               idx_v, b0, b1, b2,
               gsem0, gsem1, gsem2, wsem0, wsem1, wsem2):
    bufs = (b0, b1, b2)
    gsems = (gsem0, gsem1, gsem2)
    wsems = (wsem0, wsem1, wsem2)
    w = lax.axis_index("s") * 2 + lax.axis_index("c")
    base = w * RPW
    pltpu.sync_copy(isrc_hbm.at[pl.ds(base, RPW)], idx_v.at[0])
    pltpu.sync_copy(idst_hbm.at[pl.ds(base, RPW)], idx_v.at[1])

    def out_ref(i):
        return gs_hbm if i < NCHUNK else gd_hbm

    def idx_slice(i):
        return idx_v.at[i // NCHUNK, pl.ds((i % NCHUNK) * CH, CH)]

    def off(i):
        return base + (i % NCHUNK) * CH

    gh = [None] * (2 * NCHUNK)
    wh = [None] * (2 * NCHUNK)
    for i in range(2 * NCHUNK):
        b = i % NBUF
        if i >= NBUF:
            wh[i - NBUF].wait()
        gh[i] = pltpu.async_copy(table_hbm.at[idx_slice(i)], bufs[b], gsems[b])
        if i >= 1:
            gh[i - 1].wait()
            wh[i - 1] = pltpu.async_copy(
                bufs[(i - 1) % NBUF], out_ref(i - 1).at[pl.ds(off(i - 1), CH)],
                wsems[(i - 1) % NBUF])
    last = 2 * NCHUNK - 1
    gh[last].wait()
    wh[last] = pltpu.async_copy(bufs[last % NBUF],
                                out_ref(last).at[pl.ds(off(last), CH)],
                                wsems[last % NBUF])
    for i in range(2 * NCHUNK - NBUF, 2 * NCHUNK):
        wh[i].wait()


def _unpack(g32):
    lo = lax.bitcast_convert_type(lax.shift_left(g32, 16), jnp.float32)
    hi = lax.bitcast_convert_type(
        lax.bitwise_and(g32, jnp.int32(-65536)), jnp.float32)
    return lo.astype(jnp.bfloat16), hi.astype(jnp.bfloat16)


def _edge_body(ef_ref, gs_ref, gd_ref,
               We_ref, be_ref, ge_ref, be2_ref,
               Wc1e_ref, Wc1s_lo_ref, Wc1s_hi_ref, Wc1d_lo_ref, Wc1d_hi_ref,
               bc1_ref, gc_ref, bc_ref,
               Wc2_ref, bc2_ref, o_ref):
    ef = ef_ref[0]
    edge_h = _ln(jnp.dot(ef, We_ref[...], preferred_element_type=jnp.float32)
                 + be_ref[...], ge_ref[...], be2_ref[...])
    gs_lo, gs_hi = _unpack(gs_ref[0, 0])
    gd_lo, gd_hi = _unpack(gd_ref[0, 0])
    h_pre = (jnp.dot(edge_h.astype(jnp.bfloat16), Wc1e_ref[...],
                     preferred_element_type=jnp.float32)
             + jnp.dot(gs_lo, Wc1s_lo_ref[...], preferred_element_type=jnp.float32)
             + jnp.dot(gs_hi, Wc1s_hi_ref[...], preferred_element_type=jnp.float32)
             + jnp.dot(gd_lo, Wc1d_lo_ref[...], preferred_element_type=jnp.float32)
             + jnp.dot(gd_hi, Wc1d_hi_ref[...], preferred_element_type=jnp.float32)
             + bc1_ref[...])
    h1 = jax.nn.gelu(_ln(h_pre, gc_ref[...], bc_ref[...]))
    o_ref[0] = jnp.dot(h1, Wc2_ref[...], preferred_element_type=jnp.float32) + bc2_ref[...]


def kernel(node_feats, node_ids, edge_index, edge_feats, Wn, bn_, gn, bn2,
           We, be_, ge, be2, tpe, decay, Wc1, bc1, gc, bc, Wc2, bc2):
    del node_ids, decay
    Wc1e = Wc1[:H].astype(jnp.bfloat16)
    Wc1s = Wc1[H:2 * H].astype(jnp.bfloat16)
    Wc1d = Wc1[2 * H:].astype(jnp.bfloat16)
    Wc1s_lo, Wc1s_hi = Wc1s[:HP], Wc1s[HP:]
    Wc1d_lo, Wc1d_hi = Wc1d[:HP], Wc1d[HP:]

    table = pl.pallas_call(
        _node_body,
        grid=(1,),
        in_specs=[
            pl.BlockSpec((T, NPF, NIN), lambda i: (0, 0, 0)),
            pl.BlockSpec((T, 1, H), lambda i: (0, 0, 0)),
            pl.BlockSpec((NIN, H), lambda i: (0, 0)),
            pl.BlockSpec((H,), lambda i: (0,)),
            pl.BlockSpec((H,), lambda i: (0,)),
            pl.BlockSpec((H,), lambda i: (0,)),
        ],
        out_specs=pl.BlockSpec((T, NPF, HP), lambda i: (0, 0, 0)),
        out_shape=jax.ShapeDtypeStruct((T, NPF, HP), jnp.int32),
    )(node_feats, tpe.reshape(T, 1, H), Wn, bn_, gn, bn2)

    table = table.reshape(T * NPF, HP)
    frame_off = (jnp.arange(T, dtype=jnp.int32) * NPF)[:, None]
    idx_src = (edge_index[:, 0, :] + frame_off).reshape(NG, GROWS)
    idx_dst = (edge_index[:, 1, :] + frame_off).reshape(NG, GROWS)

    mesh = plsc.VectorSubcoreMesh(core_axis_name="c", subcore_axis_name="s")
    sc_call = pl.kernel(
        _sc_gather,
        mesh=mesh,
        out_type=(
            jax.ShapeDtypeStruct((GROWS, HP), jnp.int32),
            jax.ShapeDtypeStruct((GROWS, HP), jnp.int32),
        ),
        scratch_types=[
            pltpu.VMEM((2, RPW), jnp.int32),
            pltpu.VMEM((CH, HP), jnp.int32),
            pltpu.VMEM((CH, HP), jnp.int32),
            pltpu.VMEM((CH, HP), jnp.int32),
            pltpu.SemaphoreType.DMA,
            pltpu.SemaphoreType.DMA,
            pltpu.SemaphoreType.DMA,
            pltpu.SemaphoreType.DMA,
            pltpu.SemaphoreType.DMA,
            pltpu.SemaphoreType.DMA,
        ],
    )

    def make_tc2(g):
        return pl.pallas_call(
            _edge_body,
            grid=(TG, EC),
            in_specs=[
                pl.BlockSpec((1, ECHUNK, EIN), lambda t, e: (g * TG + t, e, 0)),
                pl.BlockSpec((1, 1, ECHUNK, HP), lambda t, e: (t, e, 0, 0)),
                pl.BlockSpec((1, 1, ECHUNK, HP), lambda t, e: (t, e, 0, 0)),
                pl.BlockSpec((EIN, H), lambda t, e: (0, 0)),
                pl.BlockSpec((H,), lambda t, e: (0,)),
                pl.BlockSpec((H,), lambda t, e: (0,)),
                pl.BlockSpec((H,), lambda t, e: (0,)),
                pl.BlockSpec((H, 2 * H), lambda t, e: (0, 0)),
                pl.BlockSpec((HP, 2 * H), lambda t, e: (0, 0)),
                pl.BlockSpec((HP, 2 * H), lambda t, e: (0, 0)),
                pl.BlockSpec((HP, 2 * H), lambda t, e: (0, 0)),
                pl.BlockSpec((HP, 2 * H), lambda t, e: (0, 0)),
                pl.BlockSpec((2 * H,), lambda t, e: (0,)),
                pl.BlockSpec((2 * H,), lambda t, e: (0,)),
                pl.BlockSpec((2 * H,), lambda t, e: (0,)),
                pl.BlockSpec((2 * H, NC), lambda t, e: (0, 0)),
                pl.BlockSpec((NC,), lambda t, e: (0,)),
            ],
            out_specs=pl.BlockSpec((1, ECHUNK, NC), lambda t, e: (t, e, 0)),
            out_shape=jax.ShapeDtypeStruct((TG, EPF, NC), jnp.float32),
            compiler_params=pltpu.CompilerParams(
                dimension_semantics=("parallel", "parallel"),
            ),
        )

    preds = []
    for g in range(NG):
        gs, gd = sc_call(table, idx_src[g], idx_dst[g])
        gs = gs.reshape(TG, EC, ECHUNK, HP)
        gd = gd.reshape(TG, EC, ECHUNK, HP)
        preds.append(make_tc2(g)(edge_feats, gs, gd, We, be_, ge, be2,
                                 Wc1e, Wc1s_lo, Wc1s_hi, Wc1d_lo, Wc1d_hi,
                                 bc1, gc, bc, Wc2, bc2))
    return jnp.concatenate(preds, axis=0)


# speedup vs baseline: 1.0596x; 1.0596x over previous
import functools

import jax
import jax.numpy as jnp
from jax import lax
from jax.experimental import pallas as pl
from jax.experimental.pallas import tpu as pltpu
from jax.experimental.pallas import tpu_sc as plsc

T = 10
NPF = 512
EPF = 4096
NIN = 256
EIN = 64
H = 256
NC = 8
HP = H // 2

NG = 2
TG = T // NG
EC = 4
ECHUNK = EPF // EC

NWORK = 32
GROWS = TG * EPF
RPW = GROWS // NWORK
CH = 128
NCHUNK = RPW // CH
NBUF = 3


def _ln(x, g, b):
    m = jnp.mean(x, axis=-1, keepdims=True)
    v = jnp.mean((x - m) ** 2, axis=-1, keepdims=True)
    return (x - m) * lax.rsqrt(v + 1e-5) * g + b


def _node_body(nf_ref, tpe_ref, Wn_ref, bn_ref, gn_ref, bn2_ref, o_ref):
    nf = nf_ref[...].reshape(T * NPF, NIN)
    node_h = _ln(jnp.dot(nf, Wn_ref[...], preferred_element_type=jnp.float32)
                 + bn_ref[...], gn_ref[...], bn2_ref[...])
    out = node_h.reshape(T, NPF, H) + tpe_ref[...]
    lo = out[:, :, :HP].astype(jnp.bfloat16).astype(jnp.float32)
    hi = out[:, :, HP:].astype(jnp.bfloat16).astype(jnp.float32)
    lo_u = lax.shift_right_logical(lax.bitcast_convert_type(lo, jnp.uint32),
                                   jnp.uint32(16))
    hi_u = lax.bitwise_and(lax.bitcast_convert_type(hi, jnp.uint32),
                           jnp.uint32(0xFFFF0000))
    o_ref[...] = lax.bitcast_convert_type(lax.bitwise_or(lo_u, hi_u), jnp.int32)


def _sc_gather(table_hbm, isrc_hbm, idst_hbm, gs_hbm, gd_hbm,
               idx_v, b0, b1, b2,
               gsem0, gsem1, gsem2, wsem0, wsem1, wsem2):
    bufs = (b0, b1, b2)
    gsems = (gsem0, gsem1, gsem2)
    wsems = (wsem0, wsem1, wsem2)
    w = lax.axis_index("s") * 2 + lax.axis_index("c")
    base = w * RPW
    pltpu.sync_copy(isrc_hbm.at[pl.ds(base, RPW)], idx_v.at[0])
    pltpu.sync_copy(idst_hbm.at[pl.ds(base, RPW)], idx_v.at[1])

    def out_ref(i):
        return gs_hbm if i < NCHUNK else gd_hbm

    def idx_slice(i):
        return idx_v.at[i // NCHUNK, pl.ds((i % NCHUNK) * CH, CH)]

    def off(i):
        return base + (i % NCHUNK) * CH

    gh = [None] * (2 * NCHUNK)
    wh = [None] * (2 * NCHUNK)
    for i in range(2 * NCHUNK):
        b = i % NBUF
        if i >= NBUF:
            wh[i - NBUF].wait()
        gh[i] = pltpu.async_copy(table_hbm.at[idx_slice(i)], bufs[b], gsems[b])
        if i >= 1:
            gh[i - 1].wait()
            wh[i - 1] = pltpu.async_copy(
                bufs[(i - 1) % NBUF], out_ref(i - 1).at[pl.ds(off(i - 1), CH)],
                wsems[(i - 1) % NBUF])
    last = 2 * NCHUNK - 1
    gh[last].wait()
    wh[last] = pltpu.async_copy(bufs[last % NBUF],
                                out_ref(last).at[pl.ds(off(last), CH)],
                                wsems[last % NBUF])
    for i in range(2 * NCHUNK - NBUF, 2 * NCHUNK):
        wh[i].wait()


def _unpack(g32):
    lo = lax.bitcast_convert_type(lax.shift_left(g32, 16), jnp.float32)
    hi = lax.bitcast_convert_type(
        lax.bitwise_and(g32, jnp.int32(-65536)), jnp.float32)
    return lo.astype(jnp.bfloat16), hi.astype(jnp.bfloat16)


def _edge_body(ef_ref, gs_ref, gd_ref,
               We_ref, be_ref, ge_ref, be2_ref,
               Wc1e_ref, Wc1s_lo_ref, Wc1s_hi_ref, Wc1d_lo_ref, Wc1d_hi_ref,
               bc1_ref, gc_ref, bc_ref,
               Wc2_ref, bc2_ref, o_ref):
    ef = ef_ref[0]
    edge_h = _ln(jnp.dot(ef, We_ref[...], preferred_element_type=jnp.float32)
                 + be_ref[...], ge_ref[...], be2_ref[...])
    gs_lo, gs_hi = _unpack(gs_ref[0, 0])
    gd_lo, gd_hi = _unpack(gd_ref[0, 0])
    h_pre = (jnp.dot(edge_h.astype(jnp.bfloat16), Wc1e_ref[...],
                     preferred_element_type=jnp.float32)
             + jnp.dot(gs_lo, Wc1s_lo_ref[...], preferred_element_type=jnp.float32)
             + jnp.dot(gs_hi, Wc1s_hi_ref[...], preferred_element_type=jnp.float32)
             + jnp.dot(gd_lo, Wc1d_lo_ref[...], preferred_element_type=jnp.float32)
             + jnp.dot(gd_hi, Wc1d_hi_ref[...], preferred_element_type=jnp.float32)
             + bc1_ref[...])
    h1 = jax.nn.gelu(_ln(h_pre, gc_ref[...], bc_ref[...]))
    o_ref[0] = jnp.dot(h1, Wc2_ref[...], preferred_element_type=jnp.float32) + bc2_ref[...]


def kernel(node_feats, node_ids, edge_index, edge_feats, Wn, bn_, gn, bn2,
           We, be_, ge, be2, tpe, decay, Wc1, bc1, gc, bc, Wc2, bc2):
    del node_ids, decay
    Wc1e = Wc1[:H].astype(jnp.bfloat16)
    Wc1s = Wc1[H:2 * H].astype(jnp.bfloat16)
    Wc1d = Wc1[2 * H:].astype(jnp.bfloat16)
    Wc1s_lo, Wc1s_hi = Wc1s[:HP], Wc1s[HP:]
    Wc1d_lo, Wc1d_hi = Wc1d[:HP], Wc1d[HP:]

    table = pl.pallas_call(
        _node_body,
        grid=(1,),
        in_specs=[
            pl.BlockSpec((T, NPF, NIN), lambda i: (0, 0, 0)),
            pl.BlockSpec((T, 1, H), lambda i: (0, 0, 0)),
            pl.BlockSpec((NIN, H), lambda i: (0, 0)),
            pl.BlockSpec((H,), lambda i: (0,)),
            pl.BlockSpec((H,), lambda i: (0,)),
            pl.BlockSpec((H,), lambda i: (0,)),
        ],
        out_specs=pl.BlockSpec((T, NPF, HP), lambda i: (0, 0, 0)),
        out_shape=jax.ShapeDtypeStruct((T, NPF, HP), jnp.int32),
    )(node_feats, tpe.reshape(T, 1, H), Wn, bn_, gn, bn2)

    table = table.reshape(T * NPF, HP)
    frame_off = (jnp.arange(T, dtype=jnp.int32) * NPF)[:, None]
    idx_src = (edge_index[:, 0, :] + frame_off).reshape(NG, GROWS)
    idx_dst = (edge_index[:, 1, :] + frame_off).reshape(NG, GROWS)

    mesh = plsc.VectorSubcoreMesh(core_axis_name="c", subcore_axis_name="s")
    sc_call = pl.kernel(
        _sc_gather,
        mesh=mesh,
        out_type=(
            jax.ShapeDtypeStruct((GROWS, HP), jnp.int32),
            jax.ShapeDtypeStruct((GROWS, HP), jnp.int32),
        ),
        scratch_types=[
            pltpu.VMEM((2, RPW), jnp.int32),
            pltpu.VMEM((CH, HP), jnp.int32),
            pltpu.VMEM((CH, HP), jnp.int32),
            pltpu.VMEM((CH, HP), jnp.int32),
            pltpu.SemaphoreType.DMA,
            pltpu.SemaphoreType.DMA,
            pltpu.SemaphoreType.DMA,
            pltpu.SemaphoreType.DMA,
            pltpu.SemaphoreType.DMA,
            pltpu.SemaphoreType.DMA,
        ],
    )

    def make_tc2(g):
        return pl.pallas_call(
            _edge_body,
            grid=(TG, EC),
            in_specs=[
                pl.BlockSpec((1, ECHUNK, EIN), lambda t, e: (g * TG + t, e, 0)),
                pl.BlockSpec((1, 1, ECHUNK, HP), lambda t, e: (t, e, 0, 0)),
                pl.BlockSpec((1, 1, ECHUNK, HP), lambda t, e: (t, e, 0, 0)),
                pl.BlockSpec((EIN, H), lambda t, e: (0, 0)),
                pl.BlockSpec((H,), lambda t, e: (0,)),
                pl.BlockSpec((H,), lambda t, e: (0,)),
                pl.BlockSpec((H,), lambda t, e: (0,)),
                pl.BlockSpec((H, 2 * H), lambda t, e: (0, 0)),
                pl.BlockSpec((HP, 2 * H), lambda t, e: (0, 0)),
                pl.BlockSpec((HP, 2 * H), lambda t, e: (0, 0)),
                pl.BlockSpec((HP, 2 * H), lambda t, e: (0, 0)),
                pl.BlockSpec((HP, 2 * H), lambda t, e: (0, 0)),
                pl.BlockSpec((2 * H,), lambda t, e: (0,)),
                pl.BlockSpec((2 * H,), lambda t, e: (0,)),
                pl.BlockSpec((2 * H,), lambda t, e: (0,)),
                pl.BlockSpec((2 * H, NC), lambda t, e: (0, 0)),
                pl.BlockSpec((NC,), lambda t, e: (0,)),
            ],
            out_specs=pl.BlockSpec((1, ECHUNK, NC), lambda t, e: (t, e, 0)),
            out_shape=jax.ShapeDtypeStruct((TG, EPF, NC), jnp.float32),
            compiler_params=pltpu.CompilerParams(
                dimension_semantics=("parallel", "parallel"),
            ),
        )

    preds = []
    for g in range(NG):
        gs, gd = sc_call(table, idx_src[g], idx_dst[g])
        gs = gs.reshape(TG, EC, ECHUNK, HP)
        gd = gd.reshape(TG, EC, ECHUNK, HP)
        preds.append(make_tc2(g)(edge_feats, gs, gd, We, be_, ge, be2,
                                 Wc1e, Wc1s_lo, Wc1s_hi, Wc1d_lo, Wc1d_hi,
                                 bc1, gc, bc, Wc2, bc2))
    return jnp.concatenate(preds, axis=0)


# speedup vs baseline: 1.1039x; 1.0418x over previous
import jax
import jax.numpy as jnp
from jax import lax
from jax.experimental import pallas as pl
from jax.experimental.pallas import tpu as pltpu
from jax.experimental.pallas import tpu_sc as plsc

T = 10
NPF = 512
EPF = 4096
NIN = 256
EIN = 64
H = 256
NC = 8
HP = H // 2

NG = 2
TG = T // NG
EC = 2
ECHUNK = EPF // EC

NWORK = 32
GROWS = TG * EPF
RPW = GROWS // NWORK
CH = 128
NCHUNK = RPW // CH
NBUF = 3


def _ln(x, g, b):
    m = jnp.mean(x, axis=-1, keepdims=True)
    v = jnp.mean((x - m) ** 2, axis=-1, keepdims=True)
    return (x - m) * lax.rsqrt(v + 1e-5) * g + b


def _node_body(nf_ref, tpe_ref, Wn_ref, bn_ref, gn_ref, bn2_ref, o_ref):
    nf = nf_ref[...].reshape(T * NPF, NIN)
    node_h = _ln(jnp.dot(nf, Wn_ref[...], preferred_element_type=jnp.float32)
                 + bn_ref[...], gn_ref[...], bn2_ref[...])
    out = node_h.reshape(T, NPF, H) + tpe_ref[...]
    lo = out[:, :, :HP].astype(jnp.bfloat16).astype(jnp.float32)
    hi = out[:, :, HP:].astype(jnp.bfloat16).astype(jnp.float32)
    lo_u = lax.shift_right_logical(lax.bitcast_convert_type(lo, jnp.uint32),
                                   jnp.uint32(16))
    hi_u = lax.bitwise_and(lax.bitcast_convert_type(hi, jnp.uint32),
                           jnp.uint32(0xFFFF0000))
    o_ref[...] = lax.bitcast_convert_type(lax.bitwise_or(lo_u, hi_u), jnp.int32)


def _sc_gather(table_hbm, isrc_hbm, idst_hbm, gs_hbm, gd_hbm,
               idx_v, b0, b1, b2,
               gsem0, gsem1, gsem2, wsem0, wsem1, wsem2):
    bufs = (b0, b1, b2)
    gsems = (gsem0, gsem1, gsem2)
    wsems = (wsem0, wsem1, wsem2)
    w = lax.axis_index("s") * 2 + lax.axis_index("c")
    base = w * RPW
    pltpu.sync_copy(isrc_hbm.at[pl.ds(base, RPW)], idx_v.at[0])
    pltpu.sync_copy(idst_hbm.at[pl.ds(base, RPW)], idx_v.at[1])

    def out_ref(i):
        return gs_hbm if i < NCHUNK else gd_hbm

    def idx_slice(i):
        return idx_v.at[i // NCHUNK, pl.ds((i % NCHUNK) * CH, CH)]

    def off(i):
        return base + (i % NCHUNK) * CH

    gh = [None] * (2 * NCHUNK)
    wh = [None] * (2 * NCHUNK)
    for i in range(2 * NCHUNK):
        b = i % NBUF
        if i >= NBUF:
            wh[i - NBUF].wait()
        gh[i] = pltpu.async_copy(table_hbm.at[idx_slice(i)], bufs[b], gsems[b])
        if i >= 1:
            gh[i - 1].wait()
            wh[i - 1] = pltpu.async_copy(
                bufs[(i - 1) % NBUF], out_ref(i - 1).at[pl.ds(off(i - 1), CH)],
                wsems[(i - 1) % NBUF])
    last = 2 * NCHUNK - 1
    gh[last].wait()
    wh[last] = pltpu.async_copy(bufs[last % NBUF],
                                out_ref(last).at[pl.ds(off(last), CH)],
                                wsems[last % NBUF])
    for i in range(2 * NCHUNK - NBUF, 2 * NCHUNK):
        wh[i].wait()


def _unpack(g32):
    lo = lax.bitcast_convert_type(lax.shift_left(g32, 16), jnp.float32)
    hi = lax.bitcast_convert_type(
        lax.bitwise_and(g32, jnp.int32(-65536)), jnp.float32)
    return lo.astype(jnp.bfloat16), hi.astype(jnp.bfloat16)


def _edge_body(ef_ref, gs_ref, gd_ref,
               We_ref, be_ref, ge_ref, be2_ref,
               Wc1e_ref, Wc1s_lo_ref, Wc1s_hi_ref, Wc1d_lo_ref, Wc1d_hi_ref,
               bc1_ref, gc_ref, bc_ref,
               Wc2_ref, bc2_ref, o_ref):
    ef = ef_ref[0]
    edge_h = _ln(jnp.dot(ef, We_ref[...], preferred_element_type=jnp.float32)
                 + be_ref[...], ge_ref[...], be2_ref[...])
    gs_lo, gs_hi = _unpack(gs_ref[0, 0])
    gd_lo, gd_hi = _unpack(gd_ref[0, 0])
    h_pre = (jnp.dot(edge_h.astype(jnp.bfloat16), Wc1e_ref[...],
                     preferred_element_type=jnp.float32)
             + jnp.dot(gs_lo, Wc1s_lo_ref[...], preferred_element_type=jnp.float32)
             + jnp.dot(gs_hi, Wc1s_hi_ref[...], preferred_element_type=jnp.float32)
             + jnp.dot(gd_lo, Wc1d_lo_ref[...], preferred_element_type=jnp.float32)
             + jnp.dot(gd_hi, Wc1d_hi_ref[...], preferred_element_type=jnp.float32)
             + bc1_ref[...])
    h1 = jax.nn.gelu(_ln(h_pre, gc_ref[...], bc_ref[...]))
    o_ref[0] = jnp.dot(h1, Wc2_ref[...], preferred_element_type=jnp.float32) + bc2_ref[...]


def kernel(node_feats, node_ids, edge_index, edge_feats, Wn, bn_, gn, bn2,
           We, be_, ge, be2, tpe, decay, Wc1, bc1, gc, bc, Wc2, bc2):
    del node_ids, decay
    Wc1e = Wc1[:H].astype(jnp.bfloat16)
    Wc1s = Wc1[H:2 * H].astype(jnp.bfloat16)
    Wc1d = Wc1[2 * H:].astype(jnp.bfloat16)
    Wc1s_lo, Wc1s_hi = Wc1s[:HP], Wc1s[HP:]
    Wc1d_lo, Wc1d_hi = Wc1d[:HP], Wc1d[HP:]

    table = pl.pallas_call(
        _node_body,
        grid=(1,),
        in_specs=[
            pl.BlockSpec((T, NPF, NIN), lambda i: (0, 0, 0)),
            pl.BlockSpec((T, 1, H), lambda i: (0, 0, 0)),
            pl.BlockSpec((NIN, H), lambda i: (0, 0)),
            pl.BlockSpec((H,), lambda i: (0,)),
            pl.BlockSpec((H,), lambda i: (0,)),
            pl.BlockSpec((H,), lambda i: (0,)),
        ],
        out_specs=pl.BlockSpec((T, NPF, HP), lambda i: (0, 0, 0)),
        out_shape=jax.ShapeDtypeStruct((T, NPF, HP), jnp.int32),
    )(node_feats, tpe.reshape(T, 1, H), Wn, bn_, gn, bn2)

    table = table.reshape(T * NPF, HP)
    frame_off = (jnp.arange(T, dtype=jnp.int32) * NPF)[:, None]
    idx_src = (edge_index[:, 0, :] + frame_off).reshape(NG, GROWS)
    idx_dst = (edge_index[:, 1, :] + frame_off).reshape(NG, GROWS)

    mesh = plsc.VectorSubcoreMesh(core_axis_name="c", subcore_axis_name="s")
    sc_call = pl.kernel(
        _sc_gather,
        mesh=mesh,
        out_type=(
            jax.ShapeDtypeStruct((GROWS, HP), jnp.int32),
            jax.ShapeDtypeStruct((GROWS, HP), jnp.int32),
        ),
        scratch_types=[
            pltpu.VMEM((2, RPW), jnp.int32),
            pltpu.VMEM((CH, HP), jnp.int32),
            pltpu.VMEM((CH, HP), jnp.int32),
            pltpu.VMEM((CH, HP), jnp.int32),
            pltpu.SemaphoreType.DMA,
            pltpu.SemaphoreType.DMA,
            pltpu.SemaphoreType.DMA,
            pltpu.SemaphoreType.DMA,
            pltpu.SemaphoreType.DMA,
            pltpu.SemaphoreType.DMA,
        ],
    )

    def make_tc2(g):
        return pl.pallas_call(
            _edge_body,
            grid=(TG, EC),
            in_specs=[
                pl.BlockSpec((1, ECHUNK, EIN), lambda t, e: (g * TG + t, e, 0)),
                pl.BlockSpec((1, 1, ECHUNK, HP), lambda t, e: (t, e, 0, 0)),
                pl.BlockSpec((1, 1, ECHUNK, HP), lambda t, e: (t, e, 0, 0)),
                pl.BlockSpec((EIN, H), lambda t, e: (0, 0)),
                pl.BlockSpec((H,), lambda t, e: (0,)),
                pl.BlockSpec((H,), lambda t, e: (0,)),
                pl.BlockSpec((H,), lambda t, e: (0,)),
                pl.BlockSpec((H, 2 * H), lambda t, e: (0, 0)),
                pl.BlockSpec((HP, 2 * H), lambda t, e: (0, 0)),
                pl.BlockSpec((HP, 2 * H), lambda t, e: (0, 0)),
                pl.BlockSpec((HP, 2 * H), lambda t, e: (0, 0)),
                pl.BlockSpec((HP, 2 * H), lambda t, e: (0, 0)),
                pl.BlockSpec((2 * H,), lambda t, e: (0,)),
                pl.BlockSpec((2 * H,), lambda t, e: (0,)),
                pl.BlockSpec((2 * H,), lambda t, e: (0,)),
                pl.BlockSpec((2 * H, NC), lambda t, e: (0, 0)),
                pl.BlockSpec((NC,), lambda t, e: (0,)),
            ],
            out_specs=pl.BlockSpec((1, ECHUNK, NC), lambda t, e: (t, e, 0)),
            out_shape=jax.ShapeDtypeStruct((TG, EPF, NC), jnp.float32),
            compiler_params=pltpu.CompilerParams(
                dimension_semantics=("parallel", "parallel"),
            ),
        )

    preds = []
    for g in range(NG):
        gs, gd = sc_call(table, idx_src[g], idx_dst[g])
        gs = gs.reshape(TG, EC, ECHUNK, HP)
        gd = gd.reshape(TG, EC, ECHUNK, HP)
        preds.append(make_tc2(g)(edge_feats, gs, gd, We, be_, ge, be2,
                                 Wc1e, Wc1s_lo, Wc1s_hi, Wc1d_lo, Wc1d_hi,
                                 bc1, gc, bc, Wc2, bc2))
    return jnp.concatenate(preds, axis=0)
